# baseline (speedup 1.0000x reference)
import functools

import jax
import jax.numpy as jnp
import numpy as np
from jax.experimental import pallas as pl
from jax.experimental.pallas import tpu as pltpu
from jax.experimental.pallas import tpu_sc as plsc

X_DIM = 2048
H_DIM = 2048
Z_DIM = 1024
TOPK = 512
B = 4
N = 4096
BLK = 256

_KERF_T = [7.853861353153693e-5, -8.010193625184903e-4, 5.188327685732524e-3,
           -2.685381193529856e-2, 1.128358514861418e-1, -3.761262582423300e-1,
           1.128379165726710e+0]
_KERFC_P = [2.326819970068386e-2, -1.387039388740657e-1, 3.687424674597105e-1,
            -5.824733027278666e-1, 6.210004621745983e-1, -4.944515323274145e-1,
            3.404879937665872e-1, -2.741127028184656e-1, 5.638259427386472e-1]
_KERFC_R = [-1.047766399936249e+1, 1.297719955372516e+1, -7.495518717768503e+0,
            2.921019019210786e+0, -1.015265279202700e+0, 4.218463358204948e-1,
            -2.820767439740514e-1, 5.641895067754075e-1]


def _erfc(x):
    t = x * x
    p = jnp.full_like(x, np.float32(0.0))
    for c in _KERF_T:
        p = p * t + np.float32(c)
    erf_small = x * p
    abs_x = jnp.abs(x)
    z = jnp.exp(-t)
    q = np.float32(1.0) / abs_x
    y = np.float32(1.0) / t
    pp = jnp.full_like(x, np.float32(0.0))
    for c in _KERFC_P:
        pp = pp * y + np.float32(c)
    pr = jnp.full_like(x, np.float32(0.0))
    for c in _KERFC_R:
        pr = pr * y + np.float32(c)
    pol = jnp.where(abs_x < np.float32(2.0), pp, pr)
    yv = (z * q) * pol
    yv = jnp.where(-t < np.float32(-88.72283905206835), np.float32(0.0), yv)
    erfc_large = jnp.where(x < np.float32(0.0), np.float32(2.0) - yv, yv)
    return jnp.where(abs_x > np.float32(1.0), erfc_large, np.float32(1.0) - erf_small)


def _gelu_exact(x):
    sqrt_half = np.sqrt(0.5).astype(np.float32)
    return 0.5 * x * _erfc(-x * sqrt_half)


def _enc_body(x_ref, W1_ref, b1_ref, g_ref, bln_ref, W2_ref, b2_ref, z_ref):
    h = jnp.dot(x_ref[...].astype(jnp.bfloat16), W1_ref[...],
                preferred_element_type=jnp.float32)
    h = h + b1_ref[...]
    h = _gelu_exact(h)
    mu = jnp.mean(h, axis=-1, keepdims=True)
    var = jnp.mean((h - mu) ** 2, axis=-1, keepdims=True)
    h = (h - mu) / jnp.sqrt(var + 1e-5) * g_ref[...] + bln_ref[...]
    z = jnp.dot(h.astype(jnp.bfloat16), W2_ref[...],
                preferred_element_type=jnp.float32)
    z_ref[...] = z + b2_ref[...]


def _encode(x2d, W1, b1, ln_g, ln_b, W2, b2):
    M = x2d.shape[0]
    return pl.pallas_call(
        _enc_body,
        grid=(M // BLK,),
        in_specs=[
            pl.BlockSpec((BLK, X_DIM), lambda i: (i, 0)),
            pl.BlockSpec((X_DIM, H_DIM), lambda i: (0, 0)),
            pl.BlockSpec((1, H_DIM), lambda i: (0, 0)),
            pl.BlockSpec((1, H_DIM), lambda i: (0, 0)),
            pl.BlockSpec((1, H_DIM), lambda i: (0, 0)),
            pl.BlockSpec((H_DIM, Z_DIM), lambda i: (0, 0)),
            pl.BlockSpec((1, Z_DIM), lambda i: (0, 0)),
        ],
        out_specs=pl.BlockSpec((BLK, Z_DIM), lambda i: (i, 0)),
        out_shape=jax.ShapeDtypeStruct((M, Z_DIM), jnp.float32),
    )(x2d, W1.astype(jnp.bfloat16), b1[None, :], ln_g[None, :], ln_b[None, :],
      W2.astype(jnp.bfloat16), b2[None, :])


def _sc_gather(z2d, idx):
    info = plsc.get_sparse_core_info()
    NC, NS = info.num_cores, info.num_subcores
    NW = NC * NS
    Bk = idx.shape[0]
    bpw = Bk // NW
    D = z2d.shape[1]
    mesh = plsc.VectorSubcoreMesh(core_axis_name="c", subcore_axis_name="s")

    @functools.partial(
        pl.kernel, mesh=mesh,
        out_type=jax.ShapeDtypeStruct((Bk, D), jnp.float32),
        scratch_types=[
            pltpu.VMEM((bpw,), jnp.int32),
            pltpu.VMEM((bpw, D), jnp.float32),
            pltpu.SemaphoreType.DMA,
        ],
    )
    def k(z_hbm, idx_hbm, out_hbm, idx_v, rows_v, sem):
        wid = jax.lax.axis_index("s") * NC + jax.lax.axis_index("c")
        base = wid * bpw
        pltpu.sync_copy(idx_hbm.at[pl.ds(base, bpw)], idx_v)
        pltpu.async_copy(z_hbm.at[idx_v], rows_v, sem).wait()
        pltpu.sync_copy(rows_v, out_hbm.at[pl.ds(base, bpw)])

    return k(z2d, idx)


def kernel(x, W1, b1, ln_g, ln_b, W2, b2, proxies, Wi1, bi1, Wi2, bi2):
    x2d = x.reshape(B * N, X_DIM)
    z2d = _encode(x2d, W1, b1, ln_g, ln_b, W2, b2)
    z_out = z2d.reshape(B, N, Z_DIM)

    h = jnp.matmul(x, W1) + b1
    h = jax.nn.gelu(h, approximate=False)
    mu = jnp.mean(h, axis=-1, keepdims=True)
    var = jnp.mean((h - mu) ** 2, axis=-1, keepdims=True)
    h = (h - mu) / jnp.sqrt(var + 1e-5) * ln_g + ln_b
    z = jnp.matmul(h, W2) + b2

    def _normalize(v, axis=-1, eps=1e-12):
        n = jnp.linalg.norm(v, axis=axis, keepdims=True)
        return v / jnp.maximum(n, eps)

    z_norm = _normalize(z, axis=-1)
    proxies_norm = _normalize(proxies, axis=-1)
    sim_matrix = jnp.matmul(z_norm, proxies_norm.T)
    imp_h = jax.nn.relu(jnp.matmul(z, Wi1) + bi1)
    importance = jax.nn.sigmoid(jnp.matmul(imp_h, Wi2) + bi2)
    weighted_sim = sim_matrix * importance
    scores = jnp.mean(weighted_sim, axis=-1)
    _, topk_indices = jax.lax.top_k(scores, TOPK)
    proxy_weights = jax.nn.softmax(jnp.mean(weighted_sim, axis=1), axis=-1)

    idx_global = (topk_indices + (jnp.arange(B, dtype=jnp.int32) * N)[:, None]).reshape(B * TOPK)
    sel = _sc_gather(z2d, idx_global).reshape(B, TOPK, Z_DIM)

    return sel, proxy_weights, topk_indices, z_out


# speedup vs baseline: 1.1991x; 1.1991x over previous
import functools

import jax
import jax.numpy as jnp
import numpy as np
from jax.experimental import pallas as pl
from jax.experimental.pallas import tpu as pltpu
from jax.experimental.pallas import tpu_sc as plsc

X_DIM = 2048
H_DIM = 2048
Z_DIM = 1024
TOPK = 512
B = 4
N = 4096
BLK = 256

_KERF_T = [7.853861353153693e-5, -8.010193625184903e-4, 5.188327685732524e-3,
           -2.685381193529856e-2, 1.128358514861418e-1, -3.761262582423300e-1,
           1.128379165726710e+0]
_KERFC_P = [2.326819970068386e-2, -1.387039388740657e-1, 3.687424674597105e-1,
            -5.824733027278666e-1, 6.210004621745983e-1, -4.944515323274145e-1,
            3.404879937665872e-1, -2.741127028184656e-1, 5.638259427386472e-1]
_KERFC_R = [-1.047766399936249e+1, 1.297719955372516e+1, -7.495518717768503e+0,
            2.921019019210786e+0, -1.015265279202700e+0, 4.218463358204948e-1,
            -2.820767439740514e-1, 5.641895067754075e-1]


def _erfc(x):
    t = x * x
    p = jnp.full_like(x, np.float32(0.0))
    for c in _KERF_T:
        p = p * t + np.float32(c)
    erf_small = x * p
    abs_x = jnp.abs(x)
    z = jnp.exp(-t)
    q = np.float32(1.0) / abs_x
    y = np.float32(1.0) / t
    pp = jnp.full_like(x, np.float32(0.0))
    for c in _KERFC_P:
        pp = pp * y + np.float32(c)
    pr = jnp.full_like(x, np.float32(0.0))
    for c in _KERFC_R:
        pr = pr * y + np.float32(c)
    pol = jnp.where(abs_x < np.float32(2.0), pp, pr)
    yv = (z * q) * pol
    yv = jnp.where(-t < np.float32(-88.72283905206835), np.float32(0.0), yv)
    erfc_large = jnp.where(x < np.float32(0.0), np.float32(2.0) - yv, yv)
    return jnp.where(abs_x > np.float32(1.0), erfc_large, np.float32(1.0) - erf_small)


def _gelu_exact(x):
    sqrt_half = np.sqrt(0.5).astype(np.float32)
    return 0.5 * x * _erfc(-x * sqrt_half)


def _enc_body(x_ref, W1_ref, b1_ref, g_ref, bln_ref, W2_ref, b2_ref, z_ref):
    h = jnp.dot(x_ref[...].astype(jnp.bfloat16), W1_ref[...],
                preferred_element_type=jnp.float32)
    h = h + b1_ref[...]
    sqrt_half = np.sqrt(0.5).astype(np.float32)
    h = 0.5 * h * (1.0 + jax.lax.erf(h * sqrt_half))
    mu = jnp.mean(h, axis=-1, keepdims=True)
    var = jnp.mean((h - mu) ** 2, axis=-1, keepdims=True)
    h = (h - mu) / jnp.sqrt(var + 1e-5) * g_ref[...] + bln_ref[...]
    z = jnp.dot(h.astype(jnp.bfloat16), W2_ref[...],
                preferred_element_type=jnp.float32)
    z_ref[...] = z + b2_ref[...]


def _encode(x2d, W1, b1, ln_g, ln_b, W2, b2):
    M = x2d.shape[0]
    return pl.pallas_call(
        _enc_body,
        grid=(M // BLK,),
        in_specs=[
            pl.BlockSpec((BLK, X_DIM), lambda i: (i, 0)),
            pl.BlockSpec((X_DIM, H_DIM), lambda i: (0, 0)),
            pl.BlockSpec((1, H_DIM), lambda i: (0, 0)),
            pl.BlockSpec((1, H_DIM), lambda i: (0, 0)),
            pl.BlockSpec((1, H_DIM), lambda i: (0, 0)),
            pl.BlockSpec((H_DIM, Z_DIM), lambda i: (0, 0)),
            pl.BlockSpec((1, Z_DIM), lambda i: (0, 0)),
        ],
        out_specs=pl.BlockSpec((BLK, Z_DIM), lambda i: (i, 0)),
        out_shape=jax.ShapeDtypeStruct((M, Z_DIM), jnp.float32),
    )(x2d, W1.astype(jnp.bfloat16), b1[None, :], ln_g[None, :], ln_b[None, :],
      W2.astype(jnp.bfloat16), b2[None, :])


def _sc_gather(z2d, idx):
    info = plsc.get_sparse_core_info()
    NC, NS = info.num_cores, info.num_subcores
    NW = NC * NS
    Bk = idx.shape[0]
    bpw = Bk // NW
    D = z2d.shape[1]
    mesh = plsc.VectorSubcoreMesh(core_axis_name="c", subcore_axis_name="s")

    @functools.partial(
        pl.kernel, mesh=mesh,
        out_type=jax.ShapeDtypeStruct((Bk, D), jnp.float32),
        scratch_types=[
            pltpu.VMEM((bpw,), jnp.int32),
            pltpu.VMEM((bpw, D), jnp.float32),
            pltpu.SemaphoreType.DMA,
        ],
    )
    def k(z_hbm, idx_hbm, out_hbm, idx_v, rows_v, sem):
        wid = jax.lax.axis_index("s") * NC + jax.lax.axis_index("c")
        base = wid * bpw
        pltpu.sync_copy(idx_hbm.at[pl.ds(base, bpw)], idx_v)
        pltpu.async_copy(z_hbm.at[idx_v], rows_v, sem).wait()
        pltpu.sync_copy(rows_v, out_hbm.at[pl.ds(base, bpw)])

    return k(z2d, idx)


def _topk_body(s_ref, o_ref):
    v = s_ref[...][0]
    e = (jax.lax.broadcasted_iota(jnp.int32, (32, 128), 0) * 128
         + jax.lax.broadcasted_iota(jnp.int32, (32, 128), 1))
    idx = e
    k = 2
    while k <= 4096:
        j = k // 2
        while j >= 1:
            bit = (e & j) != 0
            if j >= 128:
                s = j // 128

                def _sub_partner(arr, s=s):
                    parts = []
                    for g in range(0, 32, 2 * s):
                        parts.append(arr[g + s:g + 2 * s])
                        parts.append(arr[g:g + s])
                    return jnp.concatenate(parts, axis=0)

                pv = _sub_partner(v)
                pi = _sub_partner(idx)
            else:
                a_v = pltpu.roll(v, 128 - j, axis=1)
                b_v = pltpu.roll(v, j, axis=1)
                a_i = pltpu.roll(idx, 128 - j, axis=1)
                b_i = pltpu.roll(idx, j, axis=1)
                pv = jnp.where(bit, b_v, a_v)
                pi = jnp.where(bit, b_i, a_i)
            region = (e & k) == 0
            want_before = jnp.logical_not(jnp.logical_xor(jnp.logical_not(bit), region))
            cur_wins = (v > pv) | ((v == pv) & (idx < pi))
            take_cur = jnp.logical_not(jnp.logical_xor(cur_wins, want_before))
            v = jnp.where(take_cur, v, pv)
            idx = jnp.where(take_cur, idx, pi)
            j //= 2
        k *= 2
    o_ref[...] = idx[:4, :][None]


def _topk512(scores):
    s3 = scores.reshape(B, 32, 128)
    out = pl.pallas_call(
        _topk_body,
        grid=(B,),
        in_specs=[pl.BlockSpec((1, 32, 128), lambda i: (i, 0, 0))],
        out_specs=pl.BlockSpec((1, 4, 128), lambda i: (i, 0, 0)),
        out_shape=jax.ShapeDtypeStruct((B, 4, 128), jnp.int32),
    )(s3)
    return out.reshape(B, TOPK)


def kernel(x, W1, b1, ln_g, ln_b, W2, b2, proxies, Wi1, bi1, Wi2, bi2):
    x2d = x.reshape(B * N, X_DIM)
    z2d = _encode(x2d, W1, b1, ln_g, ln_b, W2, b2)
    z_out = z2d.reshape(B, N, Z_DIM)

    h = jnp.matmul(x, W1) + b1
    h = jax.nn.gelu(h, approximate=False)
    mu = jnp.mean(h, axis=-1, keepdims=True)
    var = jnp.mean((h - mu) ** 2, axis=-1, keepdims=True)
    h = (h - mu) / jnp.sqrt(var + 1e-5) * ln_g + ln_b
    z = jnp.matmul(h, W2) + b2

    def _normalize(v, axis=-1, eps=1e-12):
        n = jnp.linalg.norm(v, axis=axis, keepdims=True)
        return v / jnp.maximum(n, eps)

    z_norm = _normalize(z, axis=-1)
    proxies_norm = _normalize(proxies, axis=-1)
    sim_matrix = jnp.matmul(z_norm, proxies_norm.T)
    imp_h = jax.nn.relu(jnp.matmul(z, Wi1) + bi1)
    importance = jax.nn.sigmoid(jnp.matmul(imp_h, Wi2) + bi2)
    weighted_sim = sim_matrix * importance
    scores = jnp.mean(weighted_sim, axis=-1)
    topk_indices = _topk512(scores)
    proxy_weights = jax.nn.softmax(jnp.mean(weighted_sim, axis=1), axis=-1)

    idx_global = (topk_indices + (jnp.arange(B, dtype=jnp.int32) * N)[:, None]).reshape(B * TOPK)
    sel = _sc_gather(z2d, idx_global).reshape(B, TOPK, Z_DIM)

    return sel, proxy_weights, topk_indices, z_out


# speedup vs baseline: 1.2216x; 1.0187x over previous
import functools

import jax
import jax.numpy as jnp
import numpy as np
from jax.experimental import pallas as pl
from jax.experimental.pallas import tpu as pltpu
from jax.experimental.pallas import tpu_sc as plsc

X_DIM = 2048
H_DIM = 2048
Z_DIM = 1024
TOPK = 512
B = 4
N = 4096
BLK = 512

_KERF_T = [7.853861353153693e-5, -8.010193625184903e-4, 5.188327685732524e-3,
           -2.685381193529856e-2, 1.128358514861418e-1, -3.761262582423300e-1,
           1.128379165726710e+0]
_KERFC_P = [2.326819970068386e-2, -1.387039388740657e-1, 3.687424674597105e-1,
            -5.824733027278666e-1, 6.210004621745983e-1, -4.944515323274145e-1,
            3.404879937665872e-1, -2.741127028184656e-1, 5.638259427386472e-1]
_KERFC_R = [-1.047766399936249e+1, 1.297719955372516e+1, -7.495518717768503e+0,
            2.921019019210786e+0, -1.015265279202700e+0, 4.218463358204948e-1,
            -2.820767439740514e-1, 5.641895067754075e-1]


def _erfc(x):
    t = x * x
    p = jnp.full_like(x, np.float32(0.0))
    for c in _KERF_T:
        p = p * t + np.float32(c)
    erf_small = x * p
    abs_x = jnp.abs(x)
    z = jnp.exp(-t)
    q = np.float32(1.0) / abs_x
    y = np.float32(1.0) / t
    pp = jnp.full_like(x, np.float32(0.0))
    for c in _KERFC_P:
        pp = pp * y + np.float32(c)
    pr = jnp.full_like(x, np.float32(0.0))
    for c in _KERFC_R:
        pr = pr * y + np.float32(c)
    pol = jnp.where(abs_x < np.float32(2.0), pp, pr)
    yv = (z * q) * pol
    yv = jnp.where(-t < np.float32(-88.72283905206835), np.float32(0.0), yv)
    erfc_large = jnp.where(x < np.float32(0.0), np.float32(2.0) - yv, yv)
    return jnp.where(abs_x > np.float32(1.0), erfc_large, np.float32(1.0) - erf_small)


def _gelu_exact(x):
    sqrt_half = np.sqrt(0.5).astype(np.float32)
    return 0.5 * x * _erfc(-x * sqrt_half)


def _enc_body(x_ref, W1_ref, b1_ref, g_ref, bln_ref, W2_ref, b2_ref, z_ref):
    h = jnp.dot(x_ref[...].astype(jnp.bfloat16), W1_ref[...],
                preferred_element_type=jnp.float32)
    h = h + b1_ref[...]
    sqrt_half = np.sqrt(0.5).astype(np.float32)
    h = 0.5 * h * (1.0 + jax.lax.erf(h * sqrt_half))
    mu = jnp.mean(h, axis=-1, keepdims=True)
    var = jnp.mean((h - mu) ** 2, axis=-1, keepdims=True)
    h = (h - mu) / jnp.sqrt(var + 1e-5) * g_ref[...] + bln_ref[...]
    z = jnp.dot(h.astype(jnp.bfloat16), W2_ref[...],
                preferred_element_type=jnp.float32)
    z_ref[...] = z + b2_ref[...]


def _encode(x2d, W1, b1, ln_g, ln_b, W2, b2):
    M = x2d.shape[0]
    return pl.pallas_call(
        _enc_body,
        grid=(M // BLK,),
        in_specs=[
            pl.BlockSpec((BLK, X_DIM), lambda i: (i, 0)),
            pl.BlockSpec((X_DIM, H_DIM), lambda i: (0, 0)),
            pl.BlockSpec((1, H_DIM), lambda i: (0, 0)),
            pl.BlockSpec((1, H_DIM), lambda i: (0, 0)),
            pl.BlockSpec((1, H_DIM), lambda i: (0, 0)),
            pl.BlockSpec((H_DIM, Z_DIM), lambda i: (0, 0)),
            pl.BlockSpec((1, Z_DIM), lambda i: (0, 0)),
        ],
        out_specs=pl.BlockSpec((BLK, Z_DIM), lambda i: (i, 0)),
        out_shape=jax.ShapeDtypeStruct((M, Z_DIM), jnp.float32),
    )(x2d, W1.astype(jnp.bfloat16), b1[None, :], ln_g[None, :], ln_b[None, :],
      W2.astype(jnp.bfloat16), b2[None, :])


def _sc_gather(z2d, idx):
    info = plsc.get_sparse_core_info()
    NC, NS = info.num_cores, info.num_subcores
    NW = NC * NS
    Bk = idx.shape[0]
    bpw = Bk // NW
    D = z2d.shape[1]
    mesh = plsc.VectorSubcoreMesh(core_axis_name="c", subcore_axis_name="s")

    @functools.partial(
        pl.kernel, mesh=mesh,
        out_type=jax.ShapeDtypeStruct((Bk, D), jnp.float32),
        scratch_types=[
            pltpu.VMEM((bpw,), jnp.int32),
            pltpu.VMEM((bpw, D), jnp.float32),
            pltpu.SemaphoreType.DMA,
        ],
    )
    def k(z_hbm, idx_hbm, out_hbm, idx_v, rows_v, sem):
        wid = jax.lax.axis_index("s") * NC + jax.lax.axis_index("c")
        base = wid * bpw
        pltpu.sync_copy(idx_hbm.at[pl.ds(base, bpw)], idx_v)
        pltpu.async_copy(z_hbm.at[idx_v], rows_v, sem).wait()
        pltpu.sync_copy(rows_v, out_hbm.at[pl.ds(base, bpw)])

    return k(z2d, idx)


def _topk_body(s_ref, o_ref):
    v = s_ref[...][0]
    e = (jax.lax.broadcasted_iota(jnp.int32, (32, 128), 0) * 128
         + jax.lax.broadcasted_iota(jnp.int32, (32, 128), 1))
    idx = e
    k = 2
    while k <= 4096:
        j = k // 2
        while j >= 1:
            bit = (e & j) != 0
            if j >= 128:
                s = j // 128

                def _sub_partner(arr, s=s):
                    parts = []
                    for g in range(0, 32, 2 * s):
                        parts.append(arr[g + s:g + 2 * s])
                        parts.append(arr[g:g + s])
                    return jnp.concatenate(parts, axis=0)

                pv = _sub_partner(v)
                pi = _sub_partner(idx)
            else:
                a_v = pltpu.roll(v, 128 - j, axis=1)
                b_v = pltpu.roll(v, j, axis=1)
                a_i = pltpu.roll(idx, 128 - j, axis=1)
                b_i = pltpu.roll(idx, j, axis=1)
                pv = jnp.where(bit, b_v, a_v)
                pi = jnp.where(bit, b_i, a_i)
            region = (e & k) == 0
            want_before = jnp.logical_not(jnp.logical_xor(jnp.logical_not(bit), region))
            cur_wins = (v > pv) | ((v == pv) & (idx < pi))
            take_cur = jnp.logical_not(jnp.logical_xor(cur_wins, want_before))
            v = jnp.where(take_cur, v, pv)
            idx = jnp.where(take_cur, idx, pi)
            j //= 2
        k *= 2
    o_ref[...] = idx[:4, :][None]


def _topk512(scores):
    s3 = scores.reshape(B, 32, 128)
    out = pl.pallas_call(
        _topk_body,
        grid=(B,),
        in_specs=[pl.BlockSpec((1, 32, 128), lambda i: (i, 0, 0))],
        out_specs=pl.BlockSpec((1, 4, 128), lambda i: (i, 0, 0)),
        out_shape=jax.ShapeDtypeStruct((B, 4, 128), jnp.int32),
    )(s3)
    return out.reshape(B, TOPK)


def kernel(x, W1, b1, ln_g, ln_b, W2, b2, proxies, Wi1, bi1, Wi2, bi2):
    x2d = x.reshape(B * N, X_DIM)
    z2d = _encode(x2d, W1, b1, ln_g, ln_b, W2, b2)
    z_out = z2d.reshape(B, N, Z_DIM)

    h = jnp.matmul(x, W1) + b1
    h = jax.nn.gelu(h, approximate=False)
    mu = jnp.mean(h, axis=-1, keepdims=True)
    var = jnp.mean((h - mu) ** 2, axis=-1, keepdims=True)
    h = (h - mu) / jnp.sqrt(var + 1e-5) * ln_g + ln_b
    z = jnp.matmul(h, W2) + b2

    def _normalize(v, axis=-1, eps=1e-12):
        n = jnp.linalg.norm(v, axis=axis, keepdims=True)
        return v / jnp.maximum(n, eps)

    z_norm = _normalize(z, axis=-1)
    proxies_norm = _normalize(proxies, axis=-1)
    sim_matrix = jnp.matmul(z_norm, proxies_norm.T)
    imp_h = jax.nn.relu(jnp.matmul(z, Wi1) + bi1)
    importance = jax.nn.sigmoid(jnp.matmul(imp_h, Wi2) + bi2)
    weighted_sim = sim_matrix * importance
    scores = jnp.mean(weighted_sim, axis=-1)
    topk_indices = _topk512(scores)
    proxy_weights = jax.nn.softmax(jnp.mean(weighted_sim, axis=1), axis=-1)

    idx_global = (topk_indices + (jnp.arange(B, dtype=jnp.int32) * N)[:, None]).reshape(B * TOPK)
    sel = _sc_gather(z2d, idx_global).reshape(B, TOPK, Z_DIM)

    return sel, proxy_weights, topk_indices, z_out
